# baseline (speedup 1.0000x reference)
import functools

import jax
import jax.numpy as jnp
from jax import lax
from jax.experimental import pallas as pl
from jax.experimental.pallas import tpu as pltpu
from jax.experimental.pallas import tpu_sc as plsc

T = 2048
E = 64
H = 1024
I = 512
K = 2
TILE = 128
MAX_TILES = (T * K) // TILE + E
PADDED = MAX_TILES * TILE
NW = 32
ROWS_PER_W = PADDED // NW
GCHUNK = 48
TOK_PER_W = T // NW


def _router_body(x_ref, gw_ref, pos_ref, wts_ref, meta_ref):
    x = x_ref[...]
    gw = gw_ref[...]
    logits = lax.dot_general(x, gw, (((1,), (1,)), ((), ())),
                             preferred_element_type=jnp.float32)
    iota_e = lax.broadcasted_iota(jnp.int32, (T, E), 1)
    m1 = jnp.max(logits, axis=1, keepdims=True)
    idx1 = jnp.min(jnp.where(logits == m1, iota_e, E), axis=1, keepdims=True)
    masked = jnp.where(iota_e == idx1, -jnp.inf, logits)
    m2 = jnp.max(masked, axis=1, keepdims=True)
    idx2 = jnp.min(jnp.where(masked == m2, iota_e, E), axis=1, keepdims=True)
    e2 = jnp.exp(m2 - m1)
    wa = 1.0 / (1.0 + e2)
    wb = e2 / (1.0 + e2)

    iota_l = lax.broadcasted_iota(jnp.int32, (T, 128), 1)
    oh0 = (iota_l == idx1).astype(jnp.int32)
    oh1 = (iota_l == idx2).astype(jnp.int32)

    def cumsum_rows(a):
        d = 1
        while d < T:
            a = a + jnp.concatenate(
                [jnp.zeros((d, 128), jnp.int32), a[: T - d, :]], axis=0)
            d *= 2
        return a

    c0 = cumsum_rows(oh0)
    c1 = cumsum_rows(oh1)
    ex0 = c0 - oh0
    ex1 = c1 - oh1
    cnt0 = c0[T - 1:T, :]
    counts = cnt0 + c1[T - 1:T, :]
    ptiles = (counts + (TILE - 1)) // TILE

    def cumsum_lanes(a):
        d = 1
        while d < 128:
            a = a + jnp.concatenate(
                [jnp.zeros((1, d), jnp.int32), a[:, : 128 - d]], axis=1)
            d *= 2
        return a

    pt_inc = cumsum_lanes(ptiles)
    poff = (pt_inc - ptiles) * TILE
    pos0 = jnp.sum(oh0 * (poff + ex0), axis=1, keepdims=True)
    pos1 = jnp.sum(oh1 * (poff + cnt0 + ex1), axis=1, keepdims=True)
    pos_ref[...] = jnp.concatenate([pos0, pos1], axis=1)
    wts_ref[...] = jnp.concatenate([wa, wb], axis=1)

    iota_t = lax.broadcasted_iota(jnp.int32, (1, 128), 1)
    te = jnp.zeros((1, 128), jnp.int32)
    for e in range(E):
        te = te + jnp.where(pt_inc[0, e] <= iota_t, 1, 0)
    te = jnp.minimum(te, E - 1)
    nact = pt_inc[0, E - 1]
    row_i = lax.broadcasted_iota(jnp.int32, (8, 128), 0)
    meta = jnp.where(row_i == 0, jnp.broadcast_to(te, (8, 128)), nact)
    meta_ref[...] = meta


def _router_call(x, gw):
    return pl.pallas_call(
        _router_body,
        out_shape=(
            jax.ShapeDtypeStruct((T, K), jnp.int32),
            jax.ShapeDtypeStruct((T, K), jnp.float32),
            jax.ShapeDtypeStruct((8, 128), jnp.int32),
        ),
    )(x, gw)


def _dispatch_body(pos_hbm, wts_hbm, x_hbm, xs_hbm, ws_hbm,
                   pos_v, wts_v, inv_v, wsf_v, rowbuf0, rowbuf1, sem0, sem1):
    wid = lax.axis_index("s") * 2 + lax.axis_index("c")
    base = wid * ROWS_PER_W
    pltpu.sync_copy(pos_hbm, pos_v)
    pltpu.sync_copy(wts_hbm, wts_v)

    def zero_body(i, _):
        inv_v[pl.ds(i * 16, 16)] = jnp.zeros((16,), jnp.int32)
        wsf_v[pl.ds(i * 16, 16)] = jnp.zeros((16,), jnp.float32)
        return 0

    lax.fori_loop(0, ROWS_PER_W // 16, zero_body, 0, unroll=4)

    lane = lax.iota(jnp.int32, 16)

    def scat_body(i, _):
        idx = pos_v[pl.ds(i * 16, 16)] - base
        ok = (idx >= 0) & (idx < ROWS_PER_W)
        idxc = jnp.minimum(jnp.maximum(idx, 0), ROWS_PER_W - 1)
        tok = (lane + i * 16) & (T - 1)
        plsc.store_scatter(inv_v, [idxc], tok, mask=ok)
        w = wts_v[pl.ds(i * 16, 16)]
        plsc.store_scatter(wsf_v, [idxc], w, mask=ok)
        return 0

    lax.fori_loop(0, (T * K) // 16, scat_body, 0, unroll=8)

    pltpu.sync_copy(wsf_v, ws_hbm.at[pl.ds(base, ROWS_PER_W)])
    nch = ROWS_PER_W // GCHUNK
    bufs = (rowbuf0, rowbuf1)
    sems = (sem0, sem1)
    copies = [None, None]
    for c in range(nch + 1):
        if c < nch:
            copies[c % 2] = pltpu.async_copy(
                x_hbm.at[inv_v.at[pl.ds(c * GCHUNK, GCHUNK)]],
                bufs[c % 2], sems[c % 2])
        if c >= 1:
            p = (c - 1) % 2
            copies[p].wait()
            pltpu.sync_copy(bufs[p],
                            xs_hbm.at[pl.ds(base + (c - 1) * GCHUNK, GCHUNK)])


def _dispatch_call(pos, wts, x):
    mesh = plsc.VectorSubcoreMesh(core_axis_name="c", subcore_axis_name="s")
    fn = pl.kernel(
        _dispatch_body,
        out_type=(
            jax.ShapeDtypeStruct((PADDED, H), jnp.float32),
            jax.ShapeDtypeStruct((PADDED,), jnp.float32),
        ),
        mesh=mesh,
        scratch_types=[
            pltpu.VMEM((T * K,), jnp.int32),
            pltpu.VMEM((T * K,), jnp.float32),
            pltpu.VMEM((ROWS_PER_W,), jnp.int32),
            pltpu.VMEM((ROWS_PER_W,), jnp.float32),
            pltpu.VMEM((GCHUNK, H), jnp.float32),
            pltpu.VMEM((GCHUNK, H), jnp.float32),
            pltpu.SemaphoreType.DMA,
            pltpu.SemaphoreType.DMA,
        ],
        compiler_params=pltpu.CompilerParams(needs_layout_passes=False),
    )
    return fn(pos, wts, x)


def _gmm_body(te_ref, nact_ref, xs_ref, w1_ref, w3_ref, w2_ref, ws_ref,
              ys_ref):
    i = pl.program_id(0)

    @pl.when(i < nact_ref[0])
    def _():
        x = xs_ref[...]
        g = lax.dot_general(x, w1_ref[0], (((1,), (1,)), ((), ())),
                            preferred_element_type=jnp.float32)
        u = lax.dot_general(x, w3_ref[0], (((1,), (1,)), ((), ())),
                            preferred_element_type=jnp.float32)
        h = g * (1.0 / (1.0 + jnp.exp(-g))) * u
        y = lax.dot_general(h, w2_ref[0], (((1,), (1,)), ((), ())),
                            preferred_element_type=jnp.float32)
        ys_ref[...] = y * ws_ref[...]


def _gmm_call(te, nact, xs, w1, w3, w2, ws):
    grid_spec = pltpu.PrefetchScalarGridSpec(
        num_scalar_prefetch=2,
        grid=(MAX_TILES,),
        in_specs=[
            pl.BlockSpec((TILE, H), lambda i, te, na: (i, 0)),
            pl.BlockSpec((1, I, H), lambda i, te, na: (te[i], 0, 0)),
            pl.BlockSpec((1, I, H), lambda i, te, na: (te[i], 0, 0)),
            pl.BlockSpec((1, H, I), lambda i, te, na: (te[i], 0, 0)),
            pl.BlockSpec((TILE, 1), lambda i, te, na: (i, 0)),
        ],
        out_specs=pl.BlockSpec((TILE, H), lambda i, te, na: (i, 0)),
    )
    return pl.pallas_call(
        _gmm_body,
        grid_spec=grid_spec,
        out_shape=jax.ShapeDtypeStruct((PADDED, H), jnp.float32),
        compiler_params=pltpu.CompilerParams(
            dimension_semantics=("arbitrary",)),
    )(te, nact, xs, w1, w3, w2, ws)


CCH = 32


def _combine_body(pos_hbm, ys_hbm, out_hbm, i0_v, i1_v, buf0, buf1, sem0,
                  sem1):
    wid = lax.axis_index("s") * 2 + lax.axis_index("c")
    base = wid * TOK_PER_W
    pltpu.sync_copy(pos_hbm.at[pl.ds(base, TOK_PER_W)], i0_v)
    pltpu.sync_copy(pos_hbm.at[pl.ds(T + base, TOK_PER_W)], i1_v)
    for c in range(TOK_PER_W // CCH):
        c0 = pltpu.async_copy(
            ys_hbm.at[i0_v.at[pl.ds(c * CCH, CCH)]], buf0, sem0)
        c1 = pltpu.async_copy(
            ys_hbm.at[i1_v.at[pl.ds(c * CCH, CCH)]], buf1, sem1)
        c0.wait()
        c1.wait()

        def add_body(k, _):
            r = k // (H // 16)
            col = (k % (H // 16)) * 16
            buf0[r, pl.ds(col, 16)] = (
                buf0[r, pl.ds(col, 16)] + buf1[r, pl.ds(col, 16)])
            return 0

        lax.fori_loop(0, CCH * (H // 16), add_body, 0, unroll=4)
        pltpu.sync_copy(buf0, out_hbm.at[pl.ds(base + c * CCH, CCH)])


def _combine_call(pos, ys):
    mesh = plsc.VectorSubcoreMesh(core_axis_name="c", subcore_axis_name="s")
    fn = pl.kernel(
        _combine_body,
        out_type=jax.ShapeDtypeStruct((T, H), jnp.float32),
        mesh=mesh,
        scratch_types=[
            pltpu.VMEM((TOK_PER_W,), jnp.int32),
            pltpu.VMEM((TOK_PER_W,), jnp.int32),
            pltpu.VMEM((CCH, H), jnp.float32),
            pltpu.VMEM((CCH, H), jnp.float32),
            pltpu.SemaphoreType.DMA,
            pltpu.SemaphoreType.DMA,
        ],
        compiler_params=pltpu.CompilerParams(needs_layout_passes=False),
    )
    return fn(pos, ys)


def kernel(hidden_states, gate_w, w1, w3, w2):
    orig_shape = hidden_states.shape
    x = hidden_states.reshape(-1, H)
    pos2, wts2, meta = _router_call(x, gate_w)
    pos = jnp.concatenate([pos2[:, 0], pos2[:, 1]])
    wts = jnp.concatenate([wts2[:, 0], wts2[:, 1]])
    xs, ws = _dispatch_call(pos, wts, x)
    te = meta[0]
    nact = meta[1, 0:1]
    ys = _gmm_call(te, nact, xs, w1, w3, w2, ws.reshape(PADDED, 1))
    out = _combine_call(pos, ys)
    return out.reshape(orig_shape)


# speedup vs baseline: 2.3788x; 2.3788x over previous
import jax
import jax.numpy as jnp
from jax import lax
from jax.experimental import pallas as pl
from jax.experimental.pallas import tpu as pltpu
from jax.experimental.pallas import tpu_sc as plsc

T = 2048
E = 64
H = 1024
I = 512
K = 2
TILE = 128
MAX_TILES = (T * K) // TILE + E
PADDED = MAX_TILES * TILE
NW = 32
ROWS_PER_W = PADDED // NW
GCHUNK = 48
TOK_PER_W = T // NW


def _router_body(x_ref, gw_ref, pos_ref, wts_ref, meta_ref):
    x = x_ref[...]
    gw = gw_ref[...]
    logits = lax.dot_general(x, gw, (((1,), (1,)), ((), ())),
                             preferred_element_type=jnp.float32)
    iota_e = lax.broadcasted_iota(jnp.int32, (T, E), 1)
    m1 = jnp.max(logits, axis=1, keepdims=True)
    idx1 = jnp.min(jnp.where(logits == m1, iota_e, E), axis=1, keepdims=True)
    masked = jnp.where(iota_e == idx1, -jnp.inf, logits)
    m2 = jnp.max(masked, axis=1, keepdims=True)
    idx2 = jnp.min(jnp.where(masked == m2, iota_e, E), axis=1, keepdims=True)
    e2 = jnp.exp(m2 - m1)
    wa = 1.0 / (1.0 + e2)
    wb = e2 / (1.0 + e2)

    iota_l = lax.broadcasted_iota(jnp.int32, (T, 128), 1)
    oh0 = (iota_l == idx1).astype(jnp.int32)
    oh1 = (iota_l == idx2).astype(jnp.int32)

    def cumsum_rows(a):
        d = 1
        while d < T:
            a = a + jnp.concatenate(
                [jnp.zeros((d, 128), jnp.int32), a[: T - d, :]], axis=0)
            d *= 2
        return a

    c0 = cumsum_rows(oh0)
    c1 = cumsum_rows(oh1)
    ex0 = c0 - oh0
    ex1 = c1 - oh1
    cnt0 = c0[T - 1:T, :]
    counts = cnt0 + c1[T - 1:T, :]
    ptiles = (counts + (TILE - 1)) // TILE

    def cumsum_lanes(a):
        d = 1
        while d < 128:
            a = a + jnp.concatenate(
                [jnp.zeros((1, d), jnp.int32), a[:, : 128 - d]], axis=1)
            d *= 2
        return a

    pt_inc = cumsum_lanes(ptiles)
    poff = (pt_inc - ptiles) * TILE
    pos0 = jnp.sum(oh0 * (poff + ex0), axis=1, keepdims=True)
    pos1 = jnp.sum(oh1 * (poff + cnt0 + ex1), axis=1, keepdims=True)
    pos_ref[...] = jnp.concatenate([pos0, pos1], axis=1)
    wts_ref[...] = jnp.concatenate([wa, wb], axis=1)

    iota_t = lax.broadcasted_iota(jnp.int32, (1, 128), 1)
    te = jnp.zeros((1, 128), jnp.int32)
    for e in range(E):
        te = te + jnp.where(pt_inc[0, e] <= iota_t, 1, 0)
    te = jnp.minimum(te, E - 1)
    nact = pt_inc[0, E - 1]
    row_i = lax.broadcasted_iota(jnp.int32, (8, 128), 0)
    meta = jnp.where(row_i == 0, jnp.broadcast_to(te, (8, 128)), nact)
    meta_ref[...] = meta


def _router_call(x, gw):
    return pl.pallas_call(
        _router_body,
        out_shape=(
            jax.ShapeDtypeStruct((T, K), jnp.int32),
            jax.ShapeDtypeStruct((T, K), jnp.float32),
            jax.ShapeDtypeStruct((8, 128), jnp.int32),
        ),
    )(x, gw)


def _dispatch_body(pos_hbm, wts_hbm, x_hbm, nrows_hbm, xs_hbm, ws_hbm,
                   pos_v, wts_v, inv_v, wsf_v, rowbuf0, rowbuf1, nrows_s,
                   sem0, sem1):
    wid = lax.axis_index("s") * 2 + lax.axis_index("c")
    base = wid * ROWS_PER_W
    pltpu.sync_copy(nrows_hbm, nrows_s)
    nrows = jnp.max(nrows_s[...], axis=0)

    @pl.when(base < nrows)
    def _():
        with jax.named_scope("disp_copy"):
            pltpu.sync_copy(pos_hbm, pos_v)
            pltpu.sync_copy(wts_hbm, wts_v)

        lane = lax.iota(jnp.int32, 16)

        def zero_body(i, _):
            inv_v[pl.ds(i * 16, 16)] = (lane + i * 16 + base) & (T - 1)
            wsf_v[pl.ds(i * 16, 16)] = jnp.zeros((16,), jnp.float32)
            return 0

        with jax.named_scope("disp_zero"):
            lax.fori_loop(0, ROWS_PER_W // 16, zero_body, 0, unroll=4)

        def scat_body(i, _):
            idx = pos_v[pl.ds(i * 16, 16)] - base
            ok = (idx >= 0) & (idx < ROWS_PER_W)
            idxc = jnp.minimum(jnp.maximum(idx, 0), ROWS_PER_W - 1)
            tok = (lane + i * 16) & (T - 1)
            plsc.store_scatter(inv_v, [idxc], tok, mask=ok)
            w = wts_v[pl.ds(i * 16, 16)]
            plsc.store_scatter(wsf_v, [idxc], w, mask=ok)
            return 0

        with jax.named_scope("disp_scat"):
            lax.fori_loop(0, (T * K) // 16, scat_body, 0, unroll=8)

        with jax.named_scope("disp_gather"):
            pltpu.sync_copy(wsf_v, ws_hbm.at[pl.ds(base, ROWS_PER_W)])
            nch = ROWS_PER_W // GCHUNK
            bufs = (rowbuf0, rowbuf1)
            sems = (sem0, sem1)
            copies = [None, None]
            for c in range(nch + 1):
                if c < nch:
                    copies[c % 2] = pltpu.async_copy(
                        x_hbm.at[inv_v.at[pl.ds(c * GCHUNK, GCHUNK)]],
                        bufs[c % 2], sems[c % 2])
                if c >= 1:
                    p = (c - 1) % 2
                    copies[p].wait()
                    pltpu.sync_copy(
                        bufs[p],
                        xs_hbm.at[pl.ds(base + (c - 1) * GCHUNK, GCHUNK)])


def _dispatch_call(pos, wts, x, nrows):
    mesh = plsc.VectorSubcoreMesh(core_axis_name="c", subcore_axis_name="s")
    fn = pl.kernel(
        _dispatch_body,
        out_type=(
            jax.ShapeDtypeStruct((PADDED, H), jnp.float32),
            jax.ShapeDtypeStruct((PADDED,), jnp.float32),
        ),
        mesh=mesh,
        scratch_types=[
            pltpu.VMEM((T * K,), jnp.int32),
            pltpu.VMEM((T * K,), jnp.float32),
            pltpu.VMEM((ROWS_PER_W,), jnp.int32),
            pltpu.VMEM((ROWS_PER_W,), jnp.float32),
            pltpu.VMEM((GCHUNK, H), jnp.float32),
            pltpu.VMEM((GCHUNK, H), jnp.float32),
            pltpu.VMEM((16,), jnp.int32),
            pltpu.SemaphoreType.DMA,
            pltpu.SemaphoreType.DMA,
        ],
        compiler_params=pltpu.CompilerParams(needs_layout_passes=False),
    )
    return fn(pos, wts, x, nrows)


def _gmm_body(te_ref, nact_ref, xs_ref, w1_ref, w3_ref, w2_ref, ws_ref,
              ys_ref):
    i = pl.program_id(0)

    @pl.when(i < nact_ref[0])
    def _():
        x = xs_ref[...]
        g = lax.dot_general(x, w1_ref[0], (((1,), (1,)), ((), ())),
                            preferred_element_type=jnp.float32)
        u = lax.dot_general(x, w3_ref[0], (((1,), (1,)), ((), ())),
                            preferred_element_type=jnp.float32)
        h = g * (1.0 / (1.0 + jnp.exp(-g))) * u
        y = lax.dot_general(h, w2_ref[0], (((1,), (1,)), ((), ())),
                            preferred_element_type=jnp.float32)
        ys_ref[...] = y * ws_ref[...]


def _gmm_call(te, nact, xs, w1, w3, w2, ws):
    grid_spec = pltpu.PrefetchScalarGridSpec(
        num_scalar_prefetch=2,
        grid=(MAX_TILES,),
        in_specs=[
            pl.BlockSpec((TILE, H), lambda i, te, na: (i, 0)),
            pl.BlockSpec((1, I, H), lambda i, te, na: (te[i], 0, 0)),
            pl.BlockSpec((1, I, H), lambda i, te, na: (te[i], 0, 0)),
            pl.BlockSpec((1, H, I), lambda i, te, na: (te[i], 0, 0)),
            pl.BlockSpec((TILE, 1), lambda i, te, na: (i, 0)),
        ],
        out_specs=pl.BlockSpec((TILE, H), lambda i, te, na: (i, 0)),
    )
    return pl.pallas_call(
        _gmm_body,
        grid_spec=grid_spec,
        out_shape=jax.ShapeDtypeStruct((PADDED, H), jnp.float32),
        compiler_params=pltpu.CompilerParams(
            dimension_semantics=("arbitrary",)),
    )(te, nact, xs, w1, w3, w2, ws)


CCH = 32


def _combine_body(pos_hbm, ys_hbm, out_hbm, i0_v, i1_v, buf0, buf1, sem0,
                  sem1):
    wid = lax.axis_index("s") * 2 + lax.axis_index("c")
    base = wid * TOK_PER_W
    pltpu.sync_copy(pos_hbm.at[pl.ds(base, TOK_PER_W)], i0_v)
    pltpu.sync_copy(pos_hbm.at[pl.ds(T + base, TOK_PER_W)], i1_v)
    for c in range(TOK_PER_W // CCH):
        c0 = pltpu.async_copy(
            ys_hbm.at[i0_v.at[pl.ds(c * CCH, CCH)]], buf0, sem0)
        c1 = pltpu.async_copy(
            ys_hbm.at[i1_v.at[pl.ds(c * CCH, CCH)]], buf1, sem1)
        c0.wait()
        c1.wait()

        def add_body(k, _):
            r = k // (H // 16)
            col = (k % (H // 16)) * 16
            buf0[r, pl.ds(col, 16)] = (
                buf0[r, pl.ds(col, 16)] + buf1[r, pl.ds(col, 16)])
            return 0

        lax.fori_loop(0, CCH * (H // 16), add_body, 0, unroll=4)
        pltpu.sync_copy(buf0, out_hbm.at[pl.ds(base + c * CCH, CCH)])


def _combine_call(pos, ys):
    mesh = plsc.VectorSubcoreMesh(core_axis_name="c", subcore_axis_name="s")
    fn = pl.kernel(
        _combine_body,
        out_type=jax.ShapeDtypeStruct((T, H), jnp.float32),
        mesh=mesh,
        scratch_types=[
            pltpu.VMEM((TOK_PER_W,), jnp.int32),
            pltpu.VMEM((TOK_PER_W,), jnp.int32),
            pltpu.VMEM((CCH, H), jnp.float32),
            pltpu.VMEM((CCH, H), jnp.float32),
            pltpu.SemaphoreType.DMA,
            pltpu.SemaphoreType.DMA,
        ],
        compiler_params=pltpu.CompilerParams(needs_layout_passes=False),
    )
    return fn(pos, ys)


def kernel(hidden_states, gate_w, w1, w3, w2):
    orig_shape = hidden_states.shape
    x = hidden_states.reshape(-1, H)
    pos2, wts2, meta = _router_call(x, gate_w)
    pos = jnp.concatenate([pos2[:, 0], pos2[:, 1]])
    wts = jnp.concatenate([wts2[:, 0], wts2[:, 1]])
    nact = meta[1, 0:1]
    nrows16 = jnp.broadcast_to(nact * TILE, (16,))
    xs, ws = _dispatch_call(pos, wts, x, nrows16)
    te = meta[0]
    ys = _gmm_call(te, nact, xs, w1, w3, w2, ws.reshape(PADDED, 1))
    out = _combine_call(pos, ys)
    return out.reshape(orig_shape)
